# Initial kernel scaffold; baseline (speedup 1.0000x reference)
import jax
import jax.numpy as jnp
from jax.experimental import pallas as pl


def kernel(x, raw_alpha, levels):
    raise NotImplementedError("write your pallas kernel here")


# speedup vs baseline: 6707.1371x; 6707.1371x over previous
import functools

import jax
import jax.numpy as jnp
from jax import lax
from jax.experimental import pallas as pl
from jax.experimental.pallas import tpu as pltpu
from jax.experimental.pallas import tpu_sc as plsc

_L = 16
_NC = 2
_NS = 16
_NW = _NC * _NS
_CH = 16384
_NBUF = 2
_LUT_N = 1024


def _sc_body(x_hbm, s_hbm, lut_hbm, out_hbm,
             in0, in1, out0, out1, s_v, lut_v,
             sem_in0, sem_in1, sem_out0, sem_out1):
  n = x_hbm.shape[0]
  per = n // _NW
  nch = per // _CH

  wid = lax.axis_index("c") * _NS + lax.axis_index("s")
  base = wid * per

  ins = (in0, in1)
  outs = (out0, out1)
  sin = (sem_in0, sem_in1)
  sout = (sem_out0, sem_out1)

  pltpu.sync_copy(s_hbm, s_v)
  pltpu.sync_copy(lut_hbm, lut_v)
  s_vec = s_v[...]

  def start_in(g, b):
    pltpu.async_copy(x_hbm.at[pl.ds(base + g * _CH, _CH)], ins[b], sin[b])

  def wait_in(b):
    pltpu.make_async_copy(x_hbm.at[pl.ds(0, _CH)], ins[b], sin[b]).wait()

  def start_out(g, b):
    pltpu.async_copy(outs[b], out_hbm.at[pl.ds(base + g * _CH, _CH)], sout[b])

  def wait_out(b):
    pltpu.make_async_copy(outs[b], out_hbm.at[pl.ds(0, _CH)], sout[b]).wait()

  def compute(b):
    src = ins[b]
    dst = outs[b]

    @plsc.parallel_loop(0, _CH, step=_L, unroll=8)
    def _(off):
      v = src[pl.ds(off, _L)]
      u = v * s_vec + 256.0
      u = jnp.minimum(u, 512.0)
      u = jnp.maximum(u, 0.0)
      idx = u.astype(jnp.int32)
      dst[pl.ds(off, _L)] = plsc.load_gather(lut_v, [idx])

  for b in range(_NBUF):
    start_in(b, b)

  for g in range(_NBUF):
    b = g
    wait_in(b)
    compute(b)
    start_out(g, b)
    start_in(g + _NBUF, b)

  @pl.loop(_NBUF, nch - _NBUF, step=_NBUF)
  def _steady(g0):
    for b in range(_NBUF):
      g = g0 + b
      wait_in(b)
      wait_out(b)
      compute(b)
      start_out(g, b)
      start_in(g + _NBUF, b)

  for g in range(nch - _NBUF, nch):
    b = g % _NBUF
    wait_in(b)
    wait_out(b)
    compute(b)
    start_out(g, b)

  for b in range(_NBUF):
    wait_out(b)


@functools.lru_cache(maxsize=None)
def _build_kernel(n):
  assert n % (_NW * _CH) == 0, n
  mesh = plsc.VectorSubcoreMesh(
      core_axis_name="c", subcore_axis_name="s",
      num_cores=_NC, num_subcores=_NS)
  return pl.kernel(
      _sc_body,
      out_type=jax.ShapeDtypeStruct((n,), jnp.float32),
      mesh=mesh,
      compiler_params=pltpu.CompilerParams(needs_layout_passes=False),
      scratch_types=[
          pltpu.VMEM((_CH,), jnp.float32),
          pltpu.VMEM((_CH,), jnp.float32),
          pltpu.VMEM((_CH,), jnp.float32),
          pltpu.VMEM((_CH,), jnp.float32),
          pltpu.VMEM((_L,), jnp.float32),
          pltpu.VMEM((_LUT_N,), jnp.float32),
          pltpu.SemaphoreType.DMA,
          pltpu.SemaphoreType.DMA,
          pltpu.SemaphoreType.DMA,
          pltpu.SemaphoreType.DMA,
      ],
  )


def kernel(x, raw_alpha, levels):
  alpha = jax.nn.softplus(raw_alpha)

  t_rep = (jnp.arange(513, dtype=jnp.float32) - 255.5) * (1.0 / 256.0)
  dist = jnp.abs(t_rep[:, None] - levels[None, :])
  lut = levels[jnp.argmin(dist, axis=1)] * alpha
  lut_pad = jnp.zeros((_LUT_N,), jnp.float32).at[:513].set(lut)
  s_arr = jnp.full((_L,), 256.0 / alpha, dtype=jnp.float32)

  xf = x.reshape(-1)
  out = _build_kernel(xf.shape[0])(xf, s_arr, lut_pad)
  return out.reshape(x.shape)


# speedup vs baseline: 12427.7073x; 1.8529x over previous
import functools

import jax
import jax.numpy as jnp
from jax import lax
from jax.experimental import pallas as pl
from jax.experimental.pallas import tpu as pltpu
from jax.experimental.pallas import tpu_sc as plsc

_L = 16
_NC = 2
_NS = 16
_NW = _NC * _NS
_ROWS = 4
_NBUF = 2
_LUT_N = 1024


def _sc_body(x_hbm, s_hbm, lut_hbm, out_hbm,
             in0, in1, out0, out1, s_v, lut_v,
             sem_in0, sem_in1, sem_out0, sem_out1):
  b_, rows, cols = x_hbm.shape
  rows_per_w = (b_ * rows) // _NW
  nch = rows_per_w // _ROWS
  w_per_b = rows // rows_per_w

  wid = lax.axis_index("c") * _NS + lax.axis_index("s")
  d0 = wid // w_per_b
  row0 = (wid % w_per_b) * rows_per_w

  ins = (in0, in1)
  outs = (out0, out1)
  sin = (sem_in0, sem_in1)
  sout = (sem_out0, sem_out1)

  pltpu.sync_copy(s_hbm, s_v)
  pltpu.sync_copy(lut_hbm, lut_v)
  s_vec = s_v[...]

  def start_in(g, b):
    pltpu.async_copy(
        x_hbm.at[d0, pl.ds(row0 + g * _ROWS, _ROWS), :], ins[b], sin[b])

  def wait_in(b):
    pltpu.make_async_copy(
        x_hbm.at[0, pl.ds(0, _ROWS), :], ins[b], sin[b]).wait()

  def start_out(g, b):
    pltpu.async_copy(
        outs[b], out_hbm.at[d0, pl.ds(row0 + g * _ROWS, _ROWS), :], sout[b])

  def wait_out(b):
    pltpu.make_async_copy(
        outs[b], out_hbm.at[0, pl.ds(0, _ROWS), :], sout[b]).wait()

  def compute(b):
    src = ins[b]
    dst = outs[b]
    for j in range(_ROWS):
      @plsc.parallel_loop(0, src.shape[1], step=_L, unroll=8)
      def _(off):
        v = src[j, pl.ds(off, _L)]
        u = v * s_vec + 256.0
        u = jnp.minimum(u, 512.0)
        u = jnp.maximum(u, 0.0)
        idx = u.astype(jnp.int32)
        dst[j, pl.ds(off, _L)] = plsc.load_gather(lut_v, [idx])

  for b in range(_NBUF):
    start_in(b, b)

  for g in range(_NBUF):
    b = g
    wait_in(b)
    compute(b)
    start_out(g, b)
    start_in(g + _NBUF, b)

  @pl.loop(_NBUF, nch - _NBUF, step=_NBUF)
  def _steady(g0):
    for b in range(_NBUF):
      g = g0 + b
      wait_in(b)
      wait_out(b)
      compute(b)
      start_out(g, b)
      start_in(g + _NBUF, b)

  for g in range(nch - _NBUF, nch):
    b = g % _NBUF
    wait_in(b)
    wait_out(b)
    compute(b)
    start_out(g, b)

  for b in range(_NBUF):
    wait_out(b)


@functools.lru_cache(maxsize=None)
def _build_kernel(shape):
  b_, rows, cols = shape
  assert (b_ * rows) % _NW == 0 and ((b_ * rows) // _NW) % _ROWS == 0, shape
  mesh = plsc.VectorSubcoreMesh(
      core_axis_name="c", subcore_axis_name="s",
      num_cores=_NC, num_subcores=_NS)
  return pl.kernel(
      _sc_body,
      out_type=jax.ShapeDtypeStruct(shape, jnp.float32),
      mesh=mesh,
      compiler_params=pltpu.CompilerParams(needs_layout_passes=False),
      scratch_types=[
          pltpu.VMEM((_ROWS, cols), jnp.float32),
          pltpu.VMEM((_ROWS, cols), jnp.float32),
          pltpu.VMEM((_ROWS, cols), jnp.float32),
          pltpu.VMEM((_ROWS, cols), jnp.float32),
          pltpu.VMEM((_L,), jnp.float32),
          pltpu.VMEM((_LUT_N,), jnp.float32),
          pltpu.SemaphoreType.DMA,
          pltpu.SemaphoreType.DMA,
          pltpu.SemaphoreType.DMA,
          pltpu.SemaphoreType.DMA,
      ],
  )


def kernel(x, raw_alpha, levels):
  alpha = jax.nn.softplus(raw_alpha)

  t_rep = (jnp.arange(513, dtype=jnp.float32) - 255.5) * (1.0 / 256.0)
  dist = jnp.abs(t_rep[:, None] - levels[None, :])
  lut = levels[jnp.argmin(dist, axis=1)] * alpha
  lut_pad = jnp.zeros((_LUT_N,), jnp.float32).at[:513].set(lut)
  s_arr = jnp.full((_L,), 256.0 / alpha, dtype=jnp.float32)

  return _build_kernel(x.shape)(x, s_arr, lut_pad)
